# baseline (speedup 1.0000x reference)
import functools

import jax
import jax.numpy as jnp
from jax import lax
from jax.experimental import pallas as pl
from jax.experimental.pallas import tpu as pltpu
from jax.experimental.pallas import tpu_sc as plsc

_B, _N, _M = 16, 20000, 100
_L = 16
_TILES = 10
_TILES_LAST = 7
_ROWS = 8

_mesh = plsc.VectorSubcoreMesh(core_axis_name="c", subcore_axis_name="s")


@functools.partial(
    pl.kernel,
    mesh=_mesh,
    out_type=jax.ShapeDtypeStruct((_B, _N), jnp.int32),
    compiler_params=pltpu.CompilerParams(
        needs_layout_passes=False, skip_device_barrier=True),
    scratch_types=[
        pltpu.VMEM((_TILES, _ROWS, 128), jnp.float32),
        pltpu.VMEM((_TILES, _ROWS, 128), jnp.int32),
        pltpu.VMEM((_B, _M), jnp.int32),
        pltpu.VMEM((_TILES, _ROWS, 128), jnp.int32),
        pltpu.SemaphoreType.DMA,
        pltpu.SemaphoreType.DMA,
        pltpu.SemaphoreType.DMA,
        pltpu.SemaphoreType.DMA,
    ],
)
def _encode(samples_hbm, matches_hbm, refs_hbm, out_hbm,
            s_v, m_v, r_v, o_v, sem_s, sem_m, sem_r, sem_o):
    wid = lax.axis_index("s") * 2 + lax.axis_index("c")
    band = wid // 16
    col_w = wid % 16
    r0 = band * _ROWS
    c0 = col_w * (_TILES * 128)
    ntiles = jnp.where(col_w == 15, _TILES_LAST, _TILES)

    cp_r = pltpu.async_copy(refs_hbm, r_v, sem_r)

    def _issue(j, _):
        src = pl.ds(c0 + j * 128, 128)
        pltpu.async_copy(samples_hbm.at[pl.ds(r0, _ROWS), src], s_v.at[j], sem_s)
        pltpu.async_copy(matches_hbm.at[pl.ds(r0, _ROWS), src], m_v.at[j], sem_m)
        return ()

    lax.fori_loop(0, ntiles, _issue, ())
    cp_r.wait()

    def _tile(j, _):
        pltpu.make_async_copy(
            samples_hbm.at[pl.ds(r0, _ROWS), pl.ds(c0, 128)], s_v.at[j], sem_s
        ).wait()
        pltpu.make_async_copy(
            matches_hbm.at[pl.ds(r0, _ROWS), pl.ds(c0, 128)], m_v.at[j], sem_m
        ).wait()

        @plsc.parallel_loop(0, _ROWS * 128, _L, unroll=4)
        def _body(i):
            r = i >> 7
            sl = pl.ds(i & 127, _L)
            b_vec = jnp.full((_L,), r0 + r, jnp.int32)
            mi = jnp.clip(m_v[j, r, sl], 0, _M - 1)
            t = plsc.load_gather(r_v, [b_vec, mi]) + 1
            s = s_v[j, r, sl]
            o_v[j, r, sl] = jnp.where(s > 0.5, t,
                                      jnp.where(s < -0.5,
                                                jnp.zeros_like(t),
                                                jnp.full_like(t, -1)))

        pltpu.async_copy(
            o_v.at[j], out_hbm.at[pl.ds(r0, _ROWS), pl.ds(c0 + j * 128, 128)],
            sem_o)
        return ()

    lax.fori_loop(0, ntiles, _tile, ())

    def _drain(j, _):
        pltpu.make_async_copy(
            o_v.at[j], out_hbm.at[pl.ds(r0, _ROWS), pl.ds(c0, 128)], sem_o
        ).wait()
        return ()

    lax.fori_loop(0, ntiles, _drain, ())


def kernel(samples, matches, refs):
    return _encode(samples, matches.astype(jnp.int32), refs.astype(jnp.int32))


# speedup vs baseline: 1.0027x; 1.0027x over previous
import functools

import jax
import jax.numpy as jnp
from jax import lax
from jax.experimental import pallas as pl
from jax.experimental.pallas import tpu as pltpu
from jax.experimental.pallas import tpu_sc as plsc

_B, _N, _M = 16, 20000, 100
_L = 16
_TILES = 10
_TILES_LAST = 7
_ROWS = 8

_mesh = plsc.VectorSubcoreMesh(core_axis_name="c", subcore_axis_name="s")


@functools.partial(
    pl.kernel,
    mesh=_mesh,
    out_type=jax.ShapeDtypeStruct((_B, _N), jnp.int32),
    compiler_params=pltpu.CompilerParams(needs_layout_passes=False),
    scratch_types=[
        pltpu.VMEM((_TILES, _ROWS, 128), jnp.float32),
        pltpu.VMEM((_TILES, _ROWS, 128), jnp.int32),
        pltpu.VMEM((_B, _M), jnp.int32),
        pltpu.VMEM((_TILES, _ROWS, 128), jnp.int32),
        pltpu.SemaphoreType.DMA,
        pltpu.SemaphoreType.DMA,
        pltpu.SemaphoreType.DMA,
        pltpu.SemaphoreType.DMA,
    ],
)
def _encode(samples_hbm, matches_hbm, refs_hbm, out_hbm,
            s_v, m_v, r_v, o_v, sem_s, sem_m, sem_r, sem_o):
    wid = lax.axis_index("s") * 2 + lax.axis_index("c")
    band = wid // 16
    col_w = wid % 16
    r0 = band * _ROWS
    c0 = col_w * (_TILES * 128)
    ntiles = jnp.where(col_w == 15, _TILES_LAST, _TILES)

    cp_r = pltpu.async_copy(refs_hbm, r_v, sem_r)

    def _issue(j, _):
        src = pl.ds(c0 + j * 128, 128)
        pltpu.async_copy(samples_hbm.at[pl.ds(r0, _ROWS), src], s_v.at[j], sem_s)
        pltpu.async_copy(matches_hbm.at[pl.ds(r0, _ROWS), src], m_v.at[j], sem_m)
        return ()

    lax.fori_loop(0, ntiles, _issue, ())
    cp_r.wait()

    def _tile(j, _):
        pltpu.make_async_copy(
            samples_hbm.at[pl.ds(r0, _ROWS), pl.ds(c0, 128)], s_v.at[j], sem_s
        ).wait()
        pltpu.make_async_copy(
            matches_hbm.at[pl.ds(r0, _ROWS), pl.ds(c0, 128)], m_v.at[j], sem_m
        ).wait()

        @plsc.parallel_loop(0, _ROWS * 128, _L, unroll=8)
        def _body(i):
            r = i >> 7
            sl = pl.ds(i & 127, _L)
            b_vec = jnp.full((_L,), r0 + r, jnp.int32)
            mi = jnp.clip(m_v[j, r, sl], 0, _M - 1)
            t = plsc.load_gather(r_v, [b_vec, mi]) + 1
            s = s_v[j, r, sl]
            o_v[j, r, sl] = jnp.where(s > 0.5, t,
                                      jnp.where(s < -0.5,
                                                jnp.zeros_like(t),
                                                jnp.full_like(t, -1)))

        pltpu.async_copy(
            o_v.at[j], out_hbm.at[pl.ds(r0, _ROWS), pl.ds(c0 + j * 128, 128)],
            sem_o)
        return ()

    lax.fori_loop(0, ntiles, _tile, ())

    def _drain(j, _):
        pltpu.make_async_copy(
            o_v.at[j], out_hbm.at[pl.ds(r0, _ROWS), pl.ds(c0, 128)], sem_o
        ).wait()
        return ()

    lax.fori_loop(0, ntiles, _drain, ())


def kernel(samples, matches, refs):
    return _encode(samples, matches.astype(jnp.int32), refs.astype(jnp.int32))


# speedup vs baseline: 1.0051x; 1.0024x over previous
import functools

import jax
import jax.numpy as jnp
from jax import lax
from jax.experimental import pallas as pl
from jax.experimental.pallas import tpu as pltpu
from jax.experimental.pallas import tpu_sc as plsc

_B, _N, _M = 16, 20000, 100
_L = 16
_TILES = 10
_TILES_LAST = 7
_ROWS = 8

_mesh = plsc.VectorSubcoreMesh(core_axis_name="c", subcore_axis_name="s")


@functools.partial(
    pl.kernel,
    mesh=_mesh,
    out_type=jax.ShapeDtypeStruct((_B, _N), jnp.int32),
    compiler_params=pltpu.CompilerParams(needs_layout_passes=False),
    scratch_types=[
        pltpu.VMEM((_TILES, _ROWS, 128), jnp.float32),
        pltpu.VMEM((_TILES, _ROWS, 128), jnp.int32),
        pltpu.VMEM((_B, _M), jnp.int32),
        pltpu.VMEM((_TILES, _ROWS, 128), jnp.int32),
        pltpu.SemaphoreType.DMA,
        pltpu.SemaphoreType.DMA,
        pltpu.SemaphoreType.DMA,
        pltpu.SemaphoreType.DMA,
    ],
)
def _encode(samples_hbm, matches_hbm, refs_hbm, out_hbm,
            s_v, m_v, r_v, o_v, sem_s, sem_m, sem_r, sem_o):
    wid = lax.axis_index("s") * 2 + lax.axis_index("c")
    band = wid // 16
    col_w = wid % 16
    r0 = band * _ROWS
    c0 = col_w * (_TILES * 128)
    ntiles = jnp.where(col_w == 15, _TILES_LAST, _TILES)

    cp_r = pltpu.async_copy(refs_hbm, r_v, sem_r)

    def _issue(j, _):
        src = pl.ds(c0 + j * 128, 128)
        pltpu.async_copy(samples_hbm.at[pl.ds(r0, _ROWS), src], s_v.at[j], sem_s)
        pltpu.async_copy(matches_hbm.at[pl.ds(r0, _ROWS), src], m_v.at[j], sem_m)
        return ()

    lax.fori_loop(0, ntiles, _issue, ())
    cp_r.wait()

    @plsc.parallel_loop(0, ntiles, 1, unroll=1)
    def _tile(j):
        pltpu.make_async_copy(
            samples_hbm.at[pl.ds(r0, _ROWS), pl.ds(c0, 128)], s_v.at[j], sem_s
        ).wait()
        pltpu.make_async_copy(
            matches_hbm.at[pl.ds(r0, _ROWS), pl.ds(c0, 128)], m_v.at[j], sem_m
        ).wait()

        @plsc.parallel_loop(0, _ROWS * 128, _L, unroll=8)
        def _body(i):
            r = i >> 7
            sl = pl.ds(i & 127, _L)
            b_vec = jnp.full((_L,), r0 + r, jnp.int32)
            mi = jnp.clip(m_v[j, r, sl], 0, _M - 1)
            t = plsc.load_gather(r_v, [b_vec, mi]) + 1
            s = s_v[j, r, sl]
            o_v[j, r, sl] = jnp.where(s > 0.5, t,
                                      jnp.where(s < -0.5,
                                                jnp.zeros_like(t),
                                                jnp.full_like(t, -1)))

        pltpu.async_copy(
            o_v.at[j], out_hbm.at[pl.ds(r0, _ROWS), pl.ds(c0 + j * 128, 128)],
            sem_o)

    def _drain(j, _):
        pltpu.make_async_copy(
            o_v.at[j], out_hbm.at[pl.ds(r0, _ROWS), pl.ds(c0, 128)], sem_o
        ).wait()
        return ()

    lax.fori_loop(0, ntiles, _drain, ())


def kernel(samples, matches, refs):
    return _encode(samples, matches.astype(jnp.int32), refs.astype(jnp.int32))
